# baseline (speedup 1.0000x reference)
import functools

import jax
import jax.numpy as jnp
from jax import lax
from jax.experimental import pallas as pl
from jax.experimental.pallas import tpu as pltpu
from jax.experimental.pallas import tpu_sc as plsc

N = 10000
DEG = 32
D = 128
H = 128

NC = 2
NS = 16
NW = NC * NS
R = N * DEG
PW = R // NW
CH = 400
NCH = PW // CH


def _sigmoid_half(x):
    return jnp.tanh(x) * 0.5 + 0.5


def _fold_half(w):
    return jnp.concatenate(
        [w[:, :2 * H] * 0.5, w[:, 2 * H:3 * H], w[:, 3 * H:] * 0.5], axis=1)


def _sc_gather(table, flat_idx):
    mesh = plsc.VectorSubcoreMesh(core_axis_name="c", subcore_axis_name="s")

    @functools.partial(
        pl.kernel,
        mesh=mesh,
        out_type=jax.ShapeDtypeStruct((R, D), jnp.float32),
        scratch_types=[
            pltpu.VMEM((PW,), jnp.int32),
            pltpu.VMEM((CH, D), jnp.float32),
            pltpu.VMEM((CH, D), jnp.float32),
            pltpu.SemaphoreType.DMA,
            pltpu.SemaphoreType.DMA,
            pltpu.SemaphoreType.DMA,
            pltpu.SemaphoreType.DMA,
        ],
    )
    def k(table_hbm, idx_hbm, out_hbm, idx_v, buf0, buf1,
          gsem0, gsem1, ssem0, ssem1):
        wid = lax.axis_index("s") * NC + lax.axis_index("c")
        base = wid * PW
        pltpu.sync_copy(idx_hbm.at[pl.ds(base, PW)], idx_v)
        bufs = (buf0, buf1)
        gsems = (gsem0, gsem1)
        ssems = (ssem0, ssem1)
        gh = [None] * NCH
        sh = [None] * NCH
        gh[0] = pltpu.async_copy(
            table_hbm.at[idx_v.at[pl.ds(0, CH)]], bufs[0], gsems[0])
        for j in range(NCH):
            if j + 1 < NCH:
                if j >= 1:
                    sh[j - 1].wait()
                gh[j + 1] = pltpu.async_copy(
                    table_hbm.at[idx_v.at[pl.ds((j + 1) * CH, CH)]],
                    bufs[(j + 1) % 2], gsems[(j + 1) % 2])
            gh[j].wait()
            sh[j] = pltpu.async_copy(
                bufs[j % 2], out_hbm.at[pl.ds(base + j * CH, CH)],
                ssems[j % 2])
        sh[NCH - 2].wait()
        sh[NCH - 1].wait()

    return k(table, flat_idx)


def _lstm_tile(m_ref, w_cat, bias_g, nrows):
    h0 = jnp.zeros((nrows, H), jnp.float32)
    c0 = jnp.zeros((nrows, H), jnp.float32)
    w_x = w_cat[:D, :]
    w_h = w_cat[D:, :]
    px0 = jnp.dot(m_ref[0].astype(jnp.bfloat16), w_x,
                  preferred_element_type=jnp.float32) + bias_g

    def step(t, carry):
        h, c, px = carry
        gates = px + jnp.dot(h.astype(jnp.bfloat16), w_h,
                             preferred_element_type=jnp.float32)
        tn = jnp.minimum(t + 1, DEG - 1)
        px2 = jnp.dot(m_ref[tn].astype(jnp.bfloat16), w_x,
                      preferred_element_type=jnp.float32) + bias_g
        i = _sigmoid_half(gates[:, :H])
        f = _sigmoid_half(gates[:, H:2 * H])
        g = jnp.tanh(gates[:, 2 * H:3 * H])
        o = _sigmoid_half(gates[:, 3 * H:])
        c2 = f * c + i * g
        h2 = o * jnp.tanh(c2)
        return (h2, c2, px2)

    h, _, _ = lax.fori_loop(0, DEG, step, (h0, c0, px0))
    return h


B1 = 1000


def _layer1_call(xin, mg, w_cat, bias_g, w_sn, bias_o):
    nt = N // B1

    def body(x_ref, m_ref, wc_ref, bg_ref, wsn_ref, bo_ref, o_ref):
        h = _lstm_tile(m_ref, wc_ref[...], bg_ref[...], B1)
        xh = jnp.concatenate([x_ref[...], h], axis=1)
        o_ref[...] = (jnp.dot(xh, wsn_ref[...],
                              preferred_element_type=jnp.float32)
                      + bo_ref[...])

    return pl.pallas_call(
        body,
        grid=(nt,),
        in_specs=[
            pl.BlockSpec((B1, D), lambda i: (i, 0)),
            pl.BlockSpec((DEG, B1, D), lambda i: (0, i, 0)),
            pl.BlockSpec((2 * D, 4 * H), lambda i: (0, 0)),
            pl.BlockSpec((1, 4 * H), lambda i: (0, 0)),
            pl.BlockSpec((2 * D, H), lambda i: (0, 0)),
            pl.BlockSpec((1, H), lambda i: (0, 0)),
        ],
        out_specs=pl.BlockSpec((B1, H), lambda i: (i, 0)),
        out_shape=jax.ShapeDtypeStruct((N, H), jnp.float32),
    )(xin, mg, w_cat, bias_g, w_sn, bias_o)


def _layer2_head_call(xin, mg, w_cat, bias_g, w_sn, bias_o,
                      fc1_W, fc1_b, fc3_W, fc3_b):
    nt = N // B1

    def body(x_ref, m_ref, wc_ref, bg_ref, wsn_ref, bo_ref,
             fc1w_ref, fc1b_ref, fc3w_ref, fc3b_ref, o_ref, mx_ref):
        i = pl.program_id(0)
        h = _lstm_tile(m_ref, wc_ref[...], bg_ref[...], B1)
        xh = jnp.concatenate([x_ref[...], h], axis=1)
        h2 = (jnp.dot(xh, wsn_ref[...], preferred_element_type=jnp.float32)
              + bo_ref[...])
        tmax = jnp.max(h2, axis=0, keepdims=True)

        @pl.when(i == 0)
        def _():
            mx_ref[...] = tmax

        @pl.when(i > 0)
        def _():
            mx_ref[...] = jnp.maximum(mx_ref[...], tmax)

        @pl.when(i == nt - 1)
        def _():
            g = mx_ref[...]
            z = jnp.maximum(
                jnp.dot(g, fc1w_ref[...], preferred_element_type=jnp.float32)
                + fc1b_ref[...], 0.0)
            o_ref[...] = (jnp.dot(z, fc3w_ref[...],
                                  preferred_element_type=jnp.float32)
                          + fc3b_ref[...])

    return pl.pallas_call(
        body,
        grid=(nt,),
        in_specs=[
            pl.BlockSpec((B1, H), lambda i: (i, 0)),
            pl.BlockSpec((DEG, B1, D), lambda i: (0, i, 0)),
            pl.BlockSpec((2 * H, 4 * H), lambda i: (0, 0)),
            pl.BlockSpec((1, 4 * H), lambda i: (0, 0)),
            pl.BlockSpec((2 * H, H), lambda i: (0, 0)),
            pl.BlockSpec((1, H), lambda i: (0, 0)),
            pl.BlockSpec((H, H // 2), lambda i: (0, 0)),
            pl.BlockSpec((1, H // 2), lambda i: (0, 0)),
            pl.BlockSpec((H // 2, 1), lambda i: (0, 0)),
            pl.BlockSpec((1, 1), lambda i: (0, 0)),
        ],
        out_specs=pl.BlockSpec((1, 1), lambda i: (0, 0)),
        out_shape=jax.ShapeDtypeStruct((1, 1), jnp.float32),
        scratch_shapes=[pltpu.VMEM((1, H), jnp.float32)],
    )(xin, mg, w_cat, bias_g, w_sn, bias_o, fc1_W, fc1_b, fc3_W, fc3_b)


def kernel(x, neighbor_idx, W_self1, W_neigh1, b1, Wih1, Whh1, bih1, bhh1,
           W_self2, W_neigh2, b2, Wih2, Whh2, bih2, bhh2,
           fc1_W, fc1_b, fc3_W, fc3_b):
    bf = jnp.bfloat16
    flat_idx = neighbor_idx.astype(jnp.int32).T.reshape(-1)

    wc1 = _fold_half(jnp.concatenate([Wih1, Whh1], axis=0)).astype(bf)
    wc2 = _fold_half(jnp.concatenate([Wih2, Whh2], axis=0)).astype(bf)
    wsn1 = jnp.concatenate([W_self1, W_neigh1], axis=0)
    wsn2 = jnp.concatenate([W_self2, W_neigh2], axis=0)
    bg1 = _fold_half((bih1 + bhh1).reshape(1, 4 * H))
    bg2 = _fold_half((bih2 + bhh2).reshape(1, 4 * H))
    bo1 = b1.reshape(1, H)
    bo2 = b2.reshape(1, H)

    mg1 = _sc_gather(x, flat_idx).reshape(DEG, N, D)
    h1 = _layer1_call(x, mg1, wc1, bg1, wsn1, bo1)

    mg2 = _sc_gather(h1, flat_idx).reshape(DEG, N, H)
    out = _layer2_head_call(h1, mg2, wc2, bg2, wsn2, bo2,
                            fc1_W, fc1_b.reshape(1, H // 2),
                            fc3_W, fc3_b.reshape(1, 1))
    return out


# speedup vs baseline: 1.9289x; 1.9289x over previous
import functools

import jax
import jax.numpy as jnp
from jax import lax
from jax.experimental import pallas as pl
from jax.experimental.pallas import tpu as pltpu
from jax.experimental.pallas import tpu_sc as plsc

N = 10000
DEG = 32
D = 128
H = 128

NC = 2
NS = 16
NW = NC * NS
R = N * DEG
PW = R // NW
CH = 400
NCH = PW // CH


def _sigmoid_half(x):
    return jnp.tanh(x) * 0.5 + 0.5


def _fold_half(w):
    return jnp.concatenate(
        [w[:, :2 * H] * 0.5, w[:, 2 * H:3 * H], w[:, 3 * H:] * 0.5], axis=1)


def _sc_gather(table, flat_idx):
    mesh = plsc.VectorSubcoreMesh(core_axis_name="c", subcore_axis_name="s")

    @functools.partial(
        pl.kernel,
        mesh=mesh,
        out_type=jax.ShapeDtypeStruct((R, D), jnp.float32),
        scratch_types=[
            pltpu.VMEM((PW,), jnp.int32),
            pltpu.VMEM((CH, D), jnp.float32),
            pltpu.VMEM((CH, D), jnp.float32),
            pltpu.SemaphoreType.DMA,
            pltpu.SemaphoreType.DMA,
            pltpu.SemaphoreType.DMA,
            pltpu.SemaphoreType.DMA,
        ],
    )
    def k(table_hbm, idx_hbm, out_hbm, idx_v, buf0, buf1,
          gsem0, gsem1, ssem0, ssem1):
        wid = lax.axis_index("s") * NC + lax.axis_index("c")
        base = wid * PW
        pltpu.sync_copy(idx_hbm.at[pl.ds(base, PW)], idx_v)
        bufs = (buf0, buf1)
        gsems = (gsem0, gsem1)
        ssems = (ssem0, ssem1)
        gh = [None] * NCH
        sh = [None] * NCH
        gh[0] = pltpu.async_copy(
            table_hbm.at[idx_v.at[pl.ds(0, CH)]], bufs[0], gsems[0])
        for j in range(NCH):
            if j + 1 < NCH:
                if j >= 1:
                    sh[j - 1].wait()
                gh[j + 1] = pltpu.async_copy(
                    table_hbm.at[idx_v.at[pl.ds((j + 1) * CH, CH)]],
                    bufs[(j + 1) % 2], gsems[(j + 1) % 2])
            gh[j].wait()
            sh[j] = pltpu.async_copy(
                bufs[j % 2], out_hbm.at[pl.ds(base + j * CH, CH)],
                ssems[j % 2])
        sh[NCH - 2].wait()
        sh[NCH - 1].wait()

    return k(table, flat_idx)


def _lstm_tile(m_ref, w_cat, bias_g, nrows):
    h0 = jnp.zeros((nrows, H), jnp.float32)
    c0 = jnp.zeros((nrows, H), jnp.float32)
    unroll = 4

    def step(s, hc):
        h, c = hc
        for u in range(unroll):
            t = s * unroll + u
            xt = m_ref[t].astype(jnp.bfloat16)
            xh = jnp.concatenate([xt, h.astype(jnp.bfloat16)], axis=1)
            gates = (jnp.dot(xh, w_cat, preferred_element_type=jnp.float32)
                     + bias_g)
            i = _sigmoid_half(gates[:, :H])
            f = _sigmoid_half(gates[:, H:2 * H])
            g = jnp.tanh(gates[:, 2 * H:3 * H])
            o = _sigmoid_half(gates[:, 3 * H:])
            c = f * c + i * g
            h = o * jnp.tanh(c)
        return (h, c)

    h, _ = lax.fori_loop(0, DEG // unroll, step, (h0, c0))
    return h


B1 = 1000


def _layer1_call(xin, mg, w_cat, bias_g, w_sn, bias_o):
    nt = N // B1

    def body(x_ref, m_ref, wc_ref, bg_ref, wsn_ref, bo_ref, o_ref):
        h = _lstm_tile(m_ref, wc_ref[...], bg_ref[...], B1)
        xh = jnp.concatenate([x_ref[...], h], axis=1)
        o_ref[...] = (jnp.dot(xh, wsn_ref[...],
                              preferred_element_type=jnp.float32)
                      + bo_ref[...])

    return pl.pallas_call(
        body,
        grid=(nt,),
        in_specs=[
            pl.BlockSpec((B1, D), lambda i: (i, 0)),
            pl.BlockSpec((DEG, B1, D), lambda i: (0, i, 0)),
            pl.BlockSpec((2 * D, 4 * H), lambda i: (0, 0)),
            pl.BlockSpec((1, 4 * H), lambda i: (0, 0)),
            pl.BlockSpec((2 * D, H), lambda i: (0, 0)),
            pl.BlockSpec((1, H), lambda i: (0, 0)),
        ],
        out_specs=pl.BlockSpec((B1, H), lambda i: (i, 0)),
        out_shape=jax.ShapeDtypeStruct((N, H), jnp.float32),
    )(xin, mg, w_cat, bias_g, w_sn, bias_o)


def _layer2_head_call(xin, mg, w_cat, bias_g, w_sn, bias_o,
                      fc1_W, fc1_b, fc3_W, fc3_b):
    nt = N // B1

    def body(x_ref, m_ref, wc_ref, bg_ref, wsn_ref, bo_ref,
             fc1w_ref, fc1b_ref, fc3w_ref, fc3b_ref, o_ref, mx_ref):
        i = pl.program_id(0)
        h = _lstm_tile(m_ref, wc_ref[...], bg_ref[...], B1)
        xh = jnp.concatenate([x_ref[...], h], axis=1)
        h2 = (jnp.dot(xh, wsn_ref[...], preferred_element_type=jnp.float32)
              + bo_ref[...])
        tmax = jnp.max(h2, axis=0, keepdims=True)

        @pl.when(i == 0)
        def _():
            mx_ref[...] = tmax

        @pl.when(i > 0)
        def _():
            mx_ref[...] = jnp.maximum(mx_ref[...], tmax)

        @pl.when(i == nt - 1)
        def _():
            g = mx_ref[...]
            z = jnp.maximum(
                jnp.dot(g, fc1w_ref[...], preferred_element_type=jnp.float32)
                + fc1b_ref[...], 0.0)
            o_ref[...] = (jnp.dot(z, fc3w_ref[...],
                                  preferred_element_type=jnp.float32)
                          + fc3b_ref[...])

    return pl.pallas_call(
        body,
        grid=(nt,),
        in_specs=[
            pl.BlockSpec((B1, H), lambda i: (i, 0)),
            pl.BlockSpec((DEG, B1, D), lambda i: (0, i, 0)),
            pl.BlockSpec((2 * H, 4 * H), lambda i: (0, 0)),
            pl.BlockSpec((1, 4 * H), lambda i: (0, 0)),
            pl.BlockSpec((2 * H, H), lambda i: (0, 0)),
            pl.BlockSpec((1, H), lambda i: (0, 0)),
            pl.BlockSpec((H, H // 2), lambda i: (0, 0)),
            pl.BlockSpec((1, H // 2), lambda i: (0, 0)),
            pl.BlockSpec((H // 2, 1), lambda i: (0, 0)),
            pl.BlockSpec((1, 1), lambda i: (0, 0)),
        ],
        out_specs=pl.BlockSpec((1, 1), lambda i: (0, 0)),
        out_shape=jax.ShapeDtypeStruct((1, 1), jnp.float32),
        scratch_shapes=[pltpu.VMEM((1, H), jnp.float32)],
    )(xin, mg, w_cat, bias_g, w_sn, bias_o, fc1_W, fc1_b, fc3_W, fc3_b)


def kernel(x, neighbor_idx, W_self1, W_neigh1, b1, Wih1, Whh1, bih1, bhh1,
           W_self2, W_neigh2, b2, Wih2, Whh2, bih2, bhh2,
           fc1_W, fc1_b, fc3_W, fc3_b):
    bf = jnp.bfloat16
    flat_idx = neighbor_idx.astype(jnp.int32).T.reshape(-1)

    wc1 = _fold_half(jnp.concatenate([Wih1, Whh1], axis=0)).astype(bf)
    wc2 = _fold_half(jnp.concatenate([Wih2, Whh2], axis=0)).astype(bf)
    wsn1 = jnp.concatenate([W_self1, W_neigh1], axis=0)
    wsn2 = jnp.concatenate([W_self2, W_neigh2], axis=0)
    bg1 = _fold_half((bih1 + bhh1).reshape(1, 4 * H))
    bg2 = _fold_half((bih2 + bhh2).reshape(1, 4 * H))
    bo1 = b1.reshape(1, H)
    bo2 = b2.reshape(1, H)

    mg1 = _sc_gather(x, flat_idx).reshape(DEG, N, D)
    h1 = _layer1_call(x, mg1, wc1, bg1, wsn1, bo1)

    mg2 = _sc_gather(h1, flat_idx).reshape(DEG, N, H)
    out = _layer2_head_call(h1, mg2, wc2, bg2, wsn2, bo2,
                            fc1_W, fc1_b.reshape(1, H // 2),
                            fc3_W, fc3_b.reshape(1, 1))
    return out


# speedup vs baseline: 2.0312x; 1.0530x over previous
import functools

import jax
import jax.numpy as jnp
from jax import lax
from jax.experimental import pallas as pl
from jax.experimental.pallas import tpu as pltpu
from jax.experimental.pallas import tpu_sc as plsc

N = 10000
DEG = 32
D = 128
H = 128

NC = 2
NS = 16
NW = NC * NS
R = N * DEG
PW = R // NW
CH = 400
NCH = PW // CH


def _sigmoid_half(x):
    return jnp.tanh(x) * 0.5 + 0.5


def _fold_half(w):
    return jnp.concatenate(
        [w[:, :2 * H] * 0.5, w[:, 2 * H:3 * H], w[:, 3 * H:] * 0.5], axis=1)


def _sc_gather(table, flat_idx):
    mesh = plsc.VectorSubcoreMesh(core_axis_name="c", subcore_axis_name="s")

    @functools.partial(
        pl.kernel,
        mesh=mesh,
        out_type=jax.ShapeDtypeStruct((R, D), jnp.float32),
        scratch_types=[
            pltpu.VMEM((PW,), jnp.int32),
            pltpu.VMEM((CH, D), jnp.float32),
            pltpu.VMEM((CH, D), jnp.float32),
            pltpu.SemaphoreType.DMA,
            pltpu.SemaphoreType.DMA,
            pltpu.SemaphoreType.DMA,
            pltpu.SemaphoreType.DMA,
        ],
    )
    def k(table_hbm, idx_hbm, out_hbm, idx_v, buf0, buf1,
          gsem0, gsem1, ssem0, ssem1):
        wid = lax.axis_index("s") * NC + lax.axis_index("c")
        base = wid * PW
        pltpu.sync_copy(idx_hbm.at[pl.ds(base, PW)], idx_v)
        bufs = (buf0, buf1)
        gsems = (gsem0, gsem1)
        ssems = (ssem0, ssem1)
        gh = [None] * NCH
        sh = [None] * NCH
        gh[0] = pltpu.async_copy(
            table_hbm.at[idx_v.at[pl.ds(0, CH)]], bufs[0], gsems[0])
        for j in range(NCH):
            if j + 1 < NCH:
                if j >= 1:
                    sh[j - 1].wait()
                gh[j + 1] = pltpu.async_copy(
                    table_hbm.at[idx_v.at[pl.ds((j + 1) * CH, CH)]],
                    bufs[(j + 1) % 2], gsems[(j + 1) % 2])
            gh[j].wait()
            sh[j] = pltpu.async_copy(
                bufs[j % 2], out_hbm.at[pl.ds(base + j * CH, CH)],
                ssems[j % 2])
        sh[NCH - 2].wait()
        sh[NCH - 1].wait()

    return k(table, flat_idx)


def _lstm_tile(m_ref, w_cat, bias_g, nrows):
    h0 = jnp.zeros((nrows, H), jnp.float32)
    c0 = jnp.zeros((nrows, H), jnp.float32)
    unroll = 8

    def step(s, hc):
        h, c = hc
        for u in range(unroll):
            t = s * unroll + u
            xt = m_ref[t].astype(jnp.bfloat16)
            xh = jnp.concatenate([xt, h.astype(jnp.bfloat16)], axis=1)
            gates = (jnp.dot(xh, w_cat, preferred_element_type=jnp.float32)
                     + bias_g)
            i = _sigmoid_half(gates[:, :H])
            f = _sigmoid_half(gates[:, H:2 * H])
            g = jnp.tanh(gates[:, 2 * H:3 * H])
            o = _sigmoid_half(gates[:, 3 * H:])
            c = f * c + i * g
            h = o * jnp.tanh(c)
        return (h, c)

    h, _ = lax.fori_loop(0, DEG // unroll, step, (h0, c0))
    return h


B1 = 1000


def _layer1_call(xin, mg, w_cat, bias_g, w_sn, bias_o):
    nt = N // B1

    def body(x_ref, m_ref, wc_ref, bg_ref, wsn_ref, bo_ref, o_ref):
        h = _lstm_tile(m_ref, wc_ref[...], bg_ref[...], B1)
        xh = jnp.concatenate([x_ref[...], h], axis=1)
        o_ref[...] = (jnp.dot(xh, wsn_ref[...],
                              preferred_element_type=jnp.float32)
                      + bo_ref[...])

    return pl.pallas_call(
        body,
        grid=(nt,),
        in_specs=[
            pl.BlockSpec((B1, D), lambda i: (i, 0)),
            pl.BlockSpec((DEG, B1, D), lambda i: (0, i, 0)),
            pl.BlockSpec((2 * D, 4 * H), lambda i: (0, 0)),
            pl.BlockSpec((1, 4 * H), lambda i: (0, 0)),
            pl.BlockSpec((2 * D, H), lambda i: (0, 0)),
            pl.BlockSpec((1, H), lambda i: (0, 0)),
        ],
        out_specs=pl.BlockSpec((B1, H), lambda i: (i, 0)),
        out_shape=jax.ShapeDtypeStruct((N, H), jnp.float32),
    )(xin, mg, w_cat, bias_g, w_sn, bias_o)


def _layer2_head_call(xin, mg, w_cat, bias_g, w_sn, bias_o,
                      fc1_W, fc1_b, fc3_W, fc3_b):
    nt = N // B1

    def body(x_ref, m_ref, wc_ref, bg_ref, wsn_ref, bo_ref,
             fc1w_ref, fc1b_ref, fc3w_ref, fc3b_ref, o_ref, mx_ref):
        i = pl.program_id(0)
        h = _lstm_tile(m_ref, wc_ref[...], bg_ref[...], B1)
        xh = jnp.concatenate([x_ref[...], h], axis=1)
        h2 = (jnp.dot(xh, wsn_ref[...], preferred_element_type=jnp.float32)
              + bo_ref[...])
        tmax = jnp.max(h2, axis=0, keepdims=True)

        @pl.when(i == 0)
        def _():
            mx_ref[...] = tmax

        @pl.when(i > 0)
        def _():
            mx_ref[...] = jnp.maximum(mx_ref[...], tmax)

        @pl.when(i == nt - 1)
        def _():
            g = mx_ref[...]
            z = jnp.maximum(
                jnp.dot(g, fc1w_ref[...], preferred_element_type=jnp.float32)
                + fc1b_ref[...], 0.0)
            o_ref[...] = (jnp.dot(z, fc3w_ref[...],
                                  preferred_element_type=jnp.float32)
                          + fc3b_ref[...])

    return pl.pallas_call(
        body,
        grid=(nt,),
        in_specs=[
            pl.BlockSpec((B1, H), lambda i: (i, 0)),
            pl.BlockSpec((DEG, B1, D), lambda i: (0, i, 0)),
            pl.BlockSpec((2 * H, 4 * H), lambda i: (0, 0)),
            pl.BlockSpec((1, 4 * H), lambda i: (0, 0)),
            pl.BlockSpec((2 * H, H), lambda i: (0, 0)),
            pl.BlockSpec((1, H), lambda i: (0, 0)),
            pl.BlockSpec((H, H // 2), lambda i: (0, 0)),
            pl.BlockSpec((1, H // 2), lambda i: (0, 0)),
            pl.BlockSpec((H // 2, 1), lambda i: (0, 0)),
            pl.BlockSpec((1, 1), lambda i: (0, 0)),
        ],
        out_specs=pl.BlockSpec((1, 1), lambda i: (0, 0)),
        out_shape=jax.ShapeDtypeStruct((1, 1), jnp.float32),
        scratch_shapes=[pltpu.VMEM((1, H), jnp.float32)],
    )(xin, mg, w_cat, bias_g, w_sn, bias_o, fc1_W, fc1_b, fc3_W, fc3_b)


def kernel(x, neighbor_idx, W_self1, W_neigh1, b1, Wih1, Whh1, bih1, bhh1,
           W_self2, W_neigh2, b2, Wih2, Whh2, bih2, bhh2,
           fc1_W, fc1_b, fc3_W, fc3_b):
    bf = jnp.bfloat16
    flat_idx = neighbor_idx.astype(jnp.int32).T.reshape(-1)

    wc1 = _fold_half(jnp.concatenate([Wih1, Whh1], axis=0)).astype(bf)
    wc2 = _fold_half(jnp.concatenate([Wih2, Whh2], axis=0)).astype(bf)
    wsn1 = jnp.concatenate([W_self1, W_neigh1], axis=0)
    wsn2 = jnp.concatenate([W_self2, W_neigh2], axis=0)
    bg1 = _fold_half((bih1 + bhh1).reshape(1, 4 * H))
    bg2 = _fold_half((bih2 + bhh2).reshape(1, 4 * H))
    bo1 = b1.reshape(1, H)
    bo2 = b2.reshape(1, H)

    mg1 = _sc_gather(x, flat_idx).reshape(DEG, N, D)
    h1 = _layer1_call(x, mg1, wc1, bg1, wsn1, bo1)

    mg2 = _sc_gather(h1, flat_idx).reshape(DEG, N, H)
    out = _layer2_head_call(h1, mg2, wc2, bg2, wsn2, bo2,
                            fc1_W, fc1_b.reshape(1, H // 2),
                            fc3_W, fc3_b.reshape(1, 1))
    return out


# speedup vs baseline: 2.2373x; 1.1015x over previous
import functools

import jax
import jax.numpy as jnp
from jax import lax
from jax.experimental import pallas as pl
from jax.experimental.pallas import tpu as pltpu
from jax.experimental.pallas import tpu_sc as plsc

N = 10000
DEG = 32
D = 128
H = 128

NC = 2
NS = 16
NW = NC * NS
R = N * DEG
PW = R // NW
CH = 400
NCH = PW // CH


def _sigmoid_half(x):
    return jnp.tanh(x) * 0.5 + 0.5


def _fold_half(w):
    return jnp.concatenate(
        [w[:, :2 * H] * 0.5, w[:, 2 * H:3 * H], w[:, 3 * H:] * 0.5], axis=1)


def _sc_gather(table, flat_idx):
    mesh = plsc.VectorSubcoreMesh(core_axis_name="c", subcore_axis_name="s")

    @functools.partial(
        pl.kernel,
        mesh=mesh,
        out_type=jax.ShapeDtypeStruct((R, D), jnp.float32),
        scratch_types=[
            pltpu.VMEM((PW,), jnp.int32),
            pltpu.VMEM((CH, D), jnp.float32),
            pltpu.VMEM((CH, D), jnp.float32),
            pltpu.SemaphoreType.DMA,
            pltpu.SemaphoreType.DMA,
            pltpu.SemaphoreType.DMA,
            pltpu.SemaphoreType.DMA,
        ],
    )
    def k(table_hbm, idx_hbm, out_hbm, idx_v, buf0, buf1,
          gsem0, gsem1, ssem0, ssem1):
        wid = lax.axis_index("s") * NC + lax.axis_index("c")
        base = wid * PW
        pltpu.sync_copy(idx_hbm.at[pl.ds(base, PW)], idx_v)
        bufs = (buf0, buf1)
        gsems = (gsem0, gsem1)
        ssems = (ssem0, ssem1)
        gh = [None] * NCH
        sh = [None] * NCH
        gh[0] = pltpu.async_copy(
            table_hbm.at[idx_v.at[pl.ds(0, CH)]], bufs[0], gsems[0])
        for j in range(NCH):
            if j + 1 < NCH:
                if j >= 1:
                    sh[j - 1].wait()
                gh[j + 1] = pltpu.async_copy(
                    table_hbm.at[idx_v.at[pl.ds((j + 1) * CH, CH)]],
                    bufs[(j + 1) % 2], gsems[(j + 1) % 2])
            gh[j].wait()
            sh[j] = pltpu.async_copy(
                bufs[j % 2], out_hbm.at[pl.ds(base + j * CH, CH)],
                ssems[j % 2])
        sh[NCH - 2].wait()
        sh[NCH - 1].wait()

    return k(table, flat_idx)


def _lstm_tile(m_ref, w_cat, bias_g, nrows):
    h0 = jnp.zeros((nrows, H), jnp.float32)
    c0 = jnp.zeros((nrows, H), jnp.float32)
    unroll = 32

    def step(s, hc):
        h, c = hc
        for u in range(unroll):
            t = s * unroll + u
            xt = m_ref[t].astype(jnp.bfloat16)
            xh = jnp.concatenate([xt, h.astype(jnp.bfloat16)], axis=1)
            gates = (jnp.dot(xh, w_cat, preferred_element_type=jnp.float32)
                     + bias_g)
            i = _sigmoid_half(gates[:, :H])
            f = _sigmoid_half(gates[:, H:2 * H])
            g = jnp.tanh(gates[:, 2 * H:3 * H])
            o = _sigmoid_half(gates[:, 3 * H:])
            c = f * c + i * g
            h = o * jnp.tanh(c)
        return (h, c)

    h, _ = lax.fori_loop(0, DEG // unroll, step, (h0, c0))
    return h


B1 = 1000


def _layer1_call(xin, mg, w_cat, bias_g, w_sn, bias_o):
    nt = N // B1

    def body(x_ref, m_ref, wc_ref, bg_ref, wsn_ref, bo_ref, o_ref):
        h = _lstm_tile(m_ref, wc_ref[...], bg_ref[...], B1)
        xh = jnp.concatenate([x_ref[...], h], axis=1)
        o_ref[...] = (jnp.dot(xh, wsn_ref[...],
                              preferred_element_type=jnp.float32)
                      + bo_ref[...])

    return pl.pallas_call(
        body,
        grid=(nt,),
        in_specs=[
            pl.BlockSpec((B1, D), lambda i: (i, 0)),
            pl.BlockSpec((DEG, B1, D), lambda i: (0, i, 0)),
            pl.BlockSpec((2 * D, 4 * H), lambda i: (0, 0)),
            pl.BlockSpec((1, 4 * H), lambda i: (0, 0)),
            pl.BlockSpec((2 * D, H), lambda i: (0, 0)),
            pl.BlockSpec((1, H), lambda i: (0, 0)),
        ],
        out_specs=pl.BlockSpec((B1, H), lambda i: (i, 0)),
        out_shape=jax.ShapeDtypeStruct((N, H), jnp.float32),
    )(xin, mg, w_cat, bias_g, w_sn, bias_o)


def _layer2_head_call(xin, mg, w_cat, bias_g, w_sn, bias_o,
                      fc1_W, fc1_b, fc3_W, fc3_b):
    nt = N // B1

    def body(x_ref, m_ref, wc_ref, bg_ref, wsn_ref, bo_ref,
             fc1w_ref, fc1b_ref, fc3w_ref, fc3b_ref, o_ref, mx_ref):
        i = pl.program_id(0)
        h = _lstm_tile(m_ref, wc_ref[...], bg_ref[...], B1)
        xh = jnp.concatenate([x_ref[...], h], axis=1)
        h2 = (jnp.dot(xh, wsn_ref[...], preferred_element_type=jnp.float32)
              + bo_ref[...])
        tmax = jnp.max(h2, axis=0, keepdims=True)

        @pl.when(i == 0)
        def _():
            mx_ref[...] = tmax

        @pl.when(i > 0)
        def _():
            mx_ref[...] = jnp.maximum(mx_ref[...], tmax)

        @pl.when(i == nt - 1)
        def _():
            g = mx_ref[...]
            z = jnp.maximum(
                jnp.dot(g, fc1w_ref[...], preferred_element_type=jnp.float32)
                + fc1b_ref[...], 0.0)
            o_ref[...] = (jnp.dot(z, fc3w_ref[...],
                                  preferred_element_type=jnp.float32)
                          + fc3b_ref[...])

    return pl.pallas_call(
        body,
        grid=(nt,),
        in_specs=[
            pl.BlockSpec((B1, H), lambda i: (i, 0)),
            pl.BlockSpec((DEG, B1, D), lambda i: (0, i, 0)),
            pl.BlockSpec((2 * H, 4 * H), lambda i: (0, 0)),
            pl.BlockSpec((1, 4 * H), lambda i: (0, 0)),
            pl.BlockSpec((2 * H, H), lambda i: (0, 0)),
            pl.BlockSpec((1, H), lambda i: (0, 0)),
            pl.BlockSpec((H, H // 2), lambda i: (0, 0)),
            pl.BlockSpec((1, H // 2), lambda i: (0, 0)),
            pl.BlockSpec((H // 2, 1), lambda i: (0, 0)),
            pl.BlockSpec((1, 1), lambda i: (0, 0)),
        ],
        out_specs=pl.BlockSpec((1, 1), lambda i: (0, 0)),
        out_shape=jax.ShapeDtypeStruct((1, 1), jnp.float32),
        scratch_shapes=[pltpu.VMEM((1, H), jnp.float32)],
    )(xin, mg, w_cat, bias_g, w_sn, bias_o, fc1_W, fc1_b, fc3_W, fc3_b)


def kernel(x, neighbor_idx, W_self1, W_neigh1, b1, Wih1, Whh1, bih1, bhh1,
           W_self2, W_neigh2, b2, Wih2, Whh2, bih2, bhh2,
           fc1_W, fc1_b, fc3_W, fc3_b):
    bf = jnp.bfloat16
    flat_idx = neighbor_idx.astype(jnp.int32).T.reshape(-1)

    wc1 = _fold_half(jnp.concatenate([Wih1, Whh1], axis=0)).astype(bf)
    wc2 = _fold_half(jnp.concatenate([Wih2, Whh2], axis=0)).astype(bf)
    wsn1 = jnp.concatenate([W_self1, W_neigh1], axis=0)
    wsn2 = jnp.concatenate([W_self2, W_neigh2], axis=0)
    bg1 = _fold_half((bih1 + bhh1).reshape(1, 4 * H))
    bg2 = _fold_half((bih2 + bhh2).reshape(1, 4 * H))
    bo1 = b1.reshape(1, H)
    bo2 = b2.reshape(1, H)

    mg1 = _sc_gather(x, flat_idx).reshape(DEG, N, D)
    h1 = _layer1_call(x, mg1, wc1, bg1, wsn1, bo1)

    mg2 = _sc_gather(h1, flat_idx).reshape(DEG, N, H)
    out = _layer2_head_call(h1, mg2, wc2, bg2, wsn2, bo2,
                            fc1_W, fc1_b.reshape(1, H // 2),
                            fc3_W, fc3_b.reshape(1, 1))
    return out


# speedup vs baseline: 2.6176x; 1.1700x over previous
import functools

import jax
import jax.numpy as jnp
from jax import lax
from jax.experimental import pallas as pl
from jax.experimental.pallas import tpu as pltpu
from jax.experimental.pallas import tpu_sc as plsc

N = 10000
DEG = 32
D = 128
H = 128

NCK = 5
NCN = N // NCK
RC = NCN * DEG

NC = 2
NS = 16
NW = NC * NS
PW = RC // NW
CH = 400
NCH = PW // CH


def _sigmoid_half(x):
    return jnp.tanh(x) * 0.5 + 0.5


def _fold_half(w):
    return jnp.concatenate(
        [w[:, :2 * H] * 0.5, w[:, 2 * H:3 * H], w[:, 3 * H:] * 0.5], axis=1)


def _sc_gather(table, flat_idx):
    mesh = plsc.VectorSubcoreMesh(core_axis_name="c", subcore_axis_name="s")

    @functools.partial(
        pl.kernel,
        mesh=mesh,
        out_type=jax.ShapeDtypeStruct((RC, D), jnp.float32),
        scratch_types=[
            pltpu.VMEM((PW,), jnp.int32),
            pltpu.VMEM((CH, D), jnp.float32),
            pltpu.VMEM((CH, D), jnp.float32),
            pltpu.SemaphoreType.DMA,
            pltpu.SemaphoreType.DMA,
            pltpu.SemaphoreType.DMA,
            pltpu.SemaphoreType.DMA,
        ],
    )
    def k(table_hbm, idx_hbm, out_hbm, idx_v, buf0, buf1,
          gsem0, gsem1, ssem0, ssem1):
        wid = lax.axis_index("s") * NC + lax.axis_index("c")
        base = wid * PW
        pltpu.sync_copy(idx_hbm.at[pl.ds(base, PW)], idx_v)
        bufs = (buf0, buf1)
        gsems = (gsem0, gsem1)
        ssems = (ssem0, ssem1)
        gh = [None] * NCH
        sh = [None] * NCH
        gh[0] = pltpu.async_copy(
            table_hbm.at[idx_v.at[pl.ds(0, CH)]], bufs[0], gsems[0])
        for j in range(NCH):
            if j + 1 < NCH:
                if j >= 1:
                    sh[j - 1].wait()
                gh[j + 1] = pltpu.async_copy(
                    table_hbm.at[idx_v.at[pl.ds((j + 1) * CH, CH)]],
                    bufs[(j + 1) % 2], gsems[(j + 1) % 2])
            gh[j].wait()
            sh[j] = pltpu.async_copy(
                bufs[j % 2], out_hbm.at[pl.ds(base + j * CH, CH)],
                ssems[j % 2])
        sh[NCH - 2].wait()
        sh[NCH - 1].wait()

    return k(table, flat_idx)


def _lstm_tile(m_ref, w_cat, bias_g, nrows):
    h = jnp.zeros((nrows, H), jnp.float32)
    c = jnp.zeros((nrows, H), jnp.float32)
    for t in range(DEG):
        xt = m_ref[t].astype(jnp.bfloat16)
        xh = jnp.concatenate([xt, h.astype(jnp.bfloat16)], axis=1)
        gates = (jnp.dot(xh, w_cat, preferred_element_type=jnp.float32)
                 + bias_g)
        i = _sigmoid_half(gates[:, :H])
        f = _sigmoid_half(gates[:, H:2 * H])
        g = jnp.tanh(gates[:, 2 * H:3 * H])
        o = _sigmoid_half(gates[:, 3 * H:])
        c = f * c + i * g
        h = o * jnp.tanh(c)
    return h


B1 = 1000
NT = NCN // B1


def _layer1_chunk(xin, mg, w_cat, bias_g, w_sn, bias_o, ck):
    def body(x_ref, m_ref, wc_ref, bg_ref, wsn_ref, bo_ref, o_ref):
        h = _lstm_tile(m_ref, wc_ref[...], bg_ref[...], B1)
        xh = jnp.concatenate([x_ref[...], h], axis=1)
        o_ref[...] = (jnp.dot(xh, wsn_ref[...],
                              preferred_element_type=jnp.float32)
                      + bo_ref[...])

    return pl.pallas_call(
        body,
        grid=(NT,),
        in_specs=[
            pl.BlockSpec((B1, D), lambda i, c=ck: (c * NT + i, 0)),
            pl.BlockSpec((DEG, B1, D), lambda i: (0, i, 0)),
            pl.BlockSpec((2 * D, 4 * H), lambda i: (0, 0)),
            pl.BlockSpec((1, 4 * H), lambda i: (0, 0)),
            pl.BlockSpec((2 * D, H), lambda i: (0, 0)),
            pl.BlockSpec((1, H), lambda i: (0, 0)),
        ],
        out_specs=pl.BlockSpec((B1, H), lambda i: (i, 0)),
        out_shape=jax.ShapeDtypeStruct((NCN, H), jnp.float32),
    )(xin, mg, w_cat, bias_g, w_sn, bias_o)


def _layer2_chunk(h1, mg, w_cat, bias_g, w_sn, bias_o, ck):
    def body(x_ref, m_ref, wc_ref, bg_ref, wsn_ref, bo_ref, o_ref, mx_ref):
        i = pl.program_id(0)
        h = _lstm_tile(m_ref, wc_ref[...], bg_ref[...], B1)
        xh = jnp.concatenate([x_ref[...], h], axis=1)
        h2 = (jnp.dot(xh, wsn_ref[...], preferred_element_type=jnp.float32)
              + bo_ref[...])
        tmax = jnp.max(h2, axis=0, keepdims=True)

        @pl.when(i == 0)
        def _():
            mx_ref[...] = tmax

        @pl.when(i > 0)
        def _():
            mx_ref[...] = jnp.maximum(mx_ref[...], tmax)

        @pl.when(i == NT - 1)
        def _():
            o_ref[...] = mx_ref[...]

    return pl.pallas_call(
        body,
        grid=(NT,),
        in_specs=[
            pl.BlockSpec((B1, H), lambda i, c=ck: (c * NT + i, 0)),
            pl.BlockSpec((DEG, B1, D), lambda i: (0, i, 0)),
            pl.BlockSpec((2 * H, 4 * H), lambda i: (0, 0)),
            pl.BlockSpec((1, 4 * H), lambda i: (0, 0)),
            pl.BlockSpec((2 * H, H), lambda i: (0, 0)),
            pl.BlockSpec((1, H), lambda i: (0, 0)),
        ],
        out_specs=pl.BlockSpec((1, H), lambda i: (0, 0)),
        out_shape=jax.ShapeDtypeStruct((1, H), jnp.float32),
        scratch_shapes=[pltpu.VMEM((1, H), jnp.float32)],
    )(h1, mg, w_cat, bias_g, w_sn, bias_o)


def _head(pmax, fc1_W, fc1_b, fc3_W, fc3_b):
    def body(p_ref, fc1w_ref, fc1b_ref, fc3w_ref, fc3b_ref, o_ref):
        g = jnp.max(p_ref[...], axis=0, keepdims=True)
        z = jnp.maximum(
            jnp.dot(g, fc1w_ref[...], preferred_element_type=jnp.float32)
            + fc1b_ref[...], 0.0)
        o_ref[...] = (jnp.dot(z, fc3w_ref[...],
                              preferred_element_type=jnp.float32)
                      + fc3b_ref[...])

    return pl.pallas_call(
        body,
        out_shape=jax.ShapeDtypeStruct((1, 1), jnp.float32),
    )(pmax, fc1_W, fc1_b, fc3_W, fc3_b)


def kernel(x, neighbor_idx, W_self1, W_neigh1, b1, Wih1, Whh1, bih1, bhh1,
           W_self2, W_neigh2, b2, Wih2, Whh2, bih2, bhh2,
           fc1_W, fc1_b, fc3_W, fc3_b):
    bf = jnp.bfloat16
    idx32 = neighbor_idx.astype(jnp.int32)
    fidx = [idx32[ck * NCN:(ck + 1) * NCN].T.reshape(-1) for ck in range(NCK)]

    wc1 = _fold_half(jnp.concatenate([Wih1, Whh1], axis=0)).astype(bf)
    wc2 = _fold_half(jnp.concatenate([Wih2, Whh2], axis=0)).astype(bf)
    wsn1 = jnp.concatenate([W_self1, W_neigh1], axis=0)
    wsn2 = jnp.concatenate([W_self2, W_neigh2], axis=0)
    bg1 = _fold_half((bih1 + bhh1).reshape(1, 4 * H))
    bg2 = _fold_half((bih2 + bhh2).reshape(1, 4 * H))
    bo1 = b1.reshape(1, H)
    bo2 = b2.reshape(1, H)

    h1_parts = []
    for ck in range(NCK):
        mg = _sc_gather(x, fidx[ck]).reshape(DEG, NCN, D)
        h1_parts.append(_layer1_chunk(x, mg, wc1, bg1, wsn1, bo1, ck))
    h1 = jnp.concatenate(h1_parts, axis=0)

    pmax_parts = []
    for ck in range(NCK):
        mg = _sc_gather(h1, fidx[ck]).reshape(DEG, NCN, H)
        pmax_parts.append(_layer2_chunk(h1, mg, wc2, bg2, wsn2, bo2, ck))
    pmax = jnp.concatenate(pmax_parts, axis=0)

    return _head(pmax, fc1_W, fc1_b.reshape(1, H // 2),
                 fc3_W, fc3_b.reshape(1, 1))


# speedup vs baseline: 2.6800x; 1.0239x over previous
import functools

import jax
import jax.numpy as jnp
from jax import lax
from jax.experimental import pallas as pl
from jax.experimental.pallas import tpu as pltpu
from jax.experimental.pallas import tpu_sc as plsc

N = 10000
DEG = 32
D = 128
H = 128

NCK = 5
NCN = N // NCK
RC = NCN * DEG

NC = 2
NS = 16
NW = NC * NS
PW = RC // NW
CH = 200
NCH = PW // CH
NBUF = 4


def _sigmoid_half(x):
    return jnp.tanh(x) * 0.5 + 0.5


def _fold_half(w):
    return jnp.concatenate(
        [w[:, :2 * H] * 0.5, w[:, 2 * H:3 * H], w[:, 3 * H:] * 0.5], axis=1)


def _sc_gather(table, flat_idx):
    mesh = plsc.VectorSubcoreMesh(core_axis_name="c", subcore_axis_name="s")

    @functools.partial(
        pl.kernel,
        mesh=mesh,
        out_type=jax.ShapeDtypeStruct((RC, D), jnp.float32),
        scratch_types=(
            [pltpu.VMEM((PW,), jnp.int32)]
            + [pltpu.VMEM((CH, D), jnp.float32)] * NBUF
            + [pltpu.SemaphoreType.DMA] * (2 * NBUF)
        ),
    )
    def k(table_hbm, idx_hbm, out_hbm, idx_v, *bufsem):
        bufs = bufsem[:NBUF]
        gsems = bufsem[NBUF:2 * NBUF]
        ssems = bufsem[2 * NBUF:]
        wid = lax.axis_index("s") * NC + lax.axis_index("c")
        base = wid * PW
        pltpu.sync_copy(idx_hbm.at[pl.ds(base, PW)], idx_v)
        lkh = NBUF - 1
        gh = [None] * NCH
        sh = [None] * NCH
        for j in range(min(lkh, NCH)):
            gh[j] = pltpu.async_copy(
                table_hbm.at[idx_v.at[pl.ds(j * CH, CH)]],
                bufs[j % NBUF], gsems[j % NBUF])
        for j in range(NCH):
            jn = j + lkh
            if jn < NCH:
                if j >= 1:
                    sh[j - 1].wait()
                gh[jn] = pltpu.async_copy(
                    table_hbm.at[idx_v.at[pl.ds(jn * CH, CH)]],
                    bufs[jn % NBUF], gsems[jn % NBUF])
            gh[j].wait()
            sh[j] = pltpu.async_copy(
                bufs[j % NBUF], out_hbm.at[pl.ds(base + j * CH, CH)],
                ssems[j % NBUF])
        for j in range(max(0, NCH - lkh - 1), NCH):
            sh[j].wait()

    return k(table, flat_idx)


def _lstm_tile(m_ref, w_cat, bias_g, nrows):
    h = jnp.zeros((nrows, H), jnp.float32)
    c = jnp.zeros((nrows, H), jnp.float32)
    for t in range(DEG):
        xt = m_ref[t].astype(jnp.bfloat16)
        xh = jnp.concatenate([xt, h.astype(jnp.bfloat16)], axis=1)
        gates = (jnp.dot(xh, w_cat, preferred_element_type=jnp.float32)
                 + bias_g)
        i = _sigmoid_half(gates[:, :H])
        f = _sigmoid_half(gates[:, H:2 * H])
        g = jnp.tanh(gates[:, 2 * H:3 * H])
        o = _sigmoid_half(gates[:, 3 * H:])
        c = f * c + i * g
        h = o * jnp.tanh(c)
    return h


B1 = 1000
NT = NCN // B1


def _layer1_chunk(xin, mg, w_cat, bias_g, w_sn, bias_o, ck):
    def body(x_ref, m_ref, wc_ref, bg_ref, wsn_ref, bo_ref, o_ref):
        h = _lstm_tile(m_ref, wc_ref[...], bg_ref[...], B1)
        xh = jnp.concatenate([x_ref[...], h], axis=1)
        o_ref[...] = (jnp.dot(xh, wsn_ref[...],
                              preferred_element_type=jnp.float32)
                      + bo_ref[...])

    return pl.pallas_call(
        body,
        grid=(NT,),
        in_specs=[
            pl.BlockSpec((B1, D), lambda i, c=ck: (c * NT + i, 0)),
            pl.BlockSpec((DEG, B1, D), lambda i: (0, i, 0)),
            pl.BlockSpec((2 * D, 4 * H), lambda i: (0, 0)),
            pl.BlockSpec((1, 4 * H), lambda i: (0, 0)),
            pl.BlockSpec((2 * D, H), lambda i: (0, 0)),
            pl.BlockSpec((1, H), lambda i: (0, 0)),
        ],
        out_specs=pl.BlockSpec((B1, H), lambda i: (i, 0)),
        out_shape=jax.ShapeDtypeStruct((NCN, H), jnp.float32),
    )(xin, mg, w_cat, bias_g, w_sn, bias_o)


def _layer2_chunk(h1, mg, w_cat, bias_g, w_sn, bias_o, ck):
    def body(x_ref, m_ref, wc_ref, bg_ref, wsn_ref, bo_ref, o_ref, mx_ref):
        i = pl.program_id(0)
        h = _lstm_tile(m_ref, wc_ref[...], bg_ref[...], B1)
        xh = jnp.concatenate([x_ref[...], h], axis=1)
        h2 = (jnp.dot(xh, wsn_ref[...], preferred_element_type=jnp.float32)
              + bo_ref[...])
        tmax = jnp.max(h2, axis=0, keepdims=True)

        @pl.when(i == 0)
        def _():
            mx_ref[...] = tmax

        @pl.when(i > 0)
        def _():
            mx_ref[...] = jnp.maximum(mx_ref[...], tmax)

        @pl.when(i == NT - 1)
        def _():
            o_ref[...] = mx_ref[...]

    return pl.pallas_call(
        body,
        grid=(NT,),
        in_specs=[
            pl.BlockSpec((B1, H), lambda i, c=ck: (c * NT + i, 0)),
            pl.BlockSpec((DEG, B1, D), lambda i: (0, i, 0)),
            pl.BlockSpec((2 * H, 4 * H), lambda i: (0, 0)),
            pl.BlockSpec((1, 4 * H), lambda i: (0, 0)),
            pl.BlockSpec((2 * H, H), lambda i: (0, 0)),
            pl.BlockSpec((1, H), lambda i: (0, 0)),
        ],
        out_specs=pl.BlockSpec((1, H), lambda i: (0, 0)),
        out_shape=jax.ShapeDtypeStruct((1, H), jnp.float32),
        scratch_shapes=[pltpu.VMEM((1, H), jnp.float32)],
    )(h1, mg, w_cat, bias_g, w_sn, bias_o)


def _head(pmax, fc1_W, fc1_b, fc3_W, fc3_b):
    def body(p_ref, fc1w_ref, fc1b_ref, fc3w_ref, fc3b_ref, o_ref):
        g = jnp.max(p_ref[...], axis=0, keepdims=True)
        z = jnp.maximum(
            jnp.dot(g, fc1w_ref[...], preferred_element_type=jnp.float32)
            + fc1b_ref[...], 0.0)
        o_ref[...] = (jnp.dot(z, fc3w_ref[...],
                              preferred_element_type=jnp.float32)
                      + fc3b_ref[...])

    return pl.pallas_call(
        body,
        out_shape=jax.ShapeDtypeStruct((1, 1), jnp.float32),
    )(pmax, fc1_W, fc1_b, fc3_W, fc3_b)


def kernel(x, neighbor_idx, W_self1, W_neigh1, b1, Wih1, Whh1, bih1, bhh1,
           W_self2, W_neigh2, b2, Wih2, Whh2, bih2, bhh2,
           fc1_W, fc1_b, fc3_W, fc3_b):
    bf = jnp.bfloat16
    idx32 = neighbor_idx.astype(jnp.int32)
    fidx = [idx32[ck * NCN:(ck + 1) * NCN].T.reshape(-1) for ck in range(NCK)]

    wc1 = _fold_half(jnp.concatenate([Wih1, Whh1], axis=0)).astype(bf)
    wc2 = _fold_half(jnp.concatenate([Wih2, Whh2], axis=0)).astype(bf)
    wsn1 = jnp.concatenate([W_self1, W_neigh1], axis=0)
    wsn2 = jnp.concatenate([W_self2, W_neigh2], axis=0)
    bg1 = _fold_half((bih1 + bhh1).reshape(1, 4 * H))
    bg2 = _fold_half((bih2 + bhh2).reshape(1, 4 * H))
    bo1 = b1.reshape(1, H)
    bo2 = b2.reshape(1, H)

    h1_parts = []
    for ck in range(NCK):
        mg = _sc_gather(x, fidx[ck]).reshape(DEG, NCN, D)
        h1_parts.append(_layer1_chunk(x, mg, wc1, bg1, wsn1, bo1, ck))
    h1 = jnp.concatenate(h1_parts, axis=0)

    pmax_parts = []
    for ck in range(NCK):
        mg = _sc_gather(h1, fidx[ck]).reshape(DEG, NCN, H)
        pmax_parts.append(_layer2_chunk(h1, mg, wc2, bg2, wsn2, bo2, ck))
    pmax = jnp.concatenate(pmax_parts, axis=0)

    return _head(pmax, fc1_W, fc1_b.reshape(1, H // 2),
                 fc3_W, fc3_b.reshape(1, 1))
